# baseline (speedup 1.0000x reference)
import functools
import math

import jax
import jax.numpy as jnp
from jax import lax
from jax.experimental import pallas as pl
from jax.experimental.pallas import tpu as pltpu
from jax.experimental.pallas import tpu_sc as plsc

N_TOKEN = 1000000
D_MODEL = 64
SCALE = math.sqrt(D_MODEL)

_info = plsc.get_sparse_core_info()
_NC, _NS, _L = _info.num_cores, _info.num_subcores, _info.num_lanes
_NW = _NC * _NS

CHUNK = 512


def _make_gather(B: int, D: int):
    assert B % (_NW * CHUNK) == 0
    b_per_w = B // _NW
    n_chunks = b_per_w // CHUNK
    mesh = plsc.VectorSubcoreMesh(core_axis_name="c", subcore_axis_name="s")

    @functools.partial(
        pl.kernel,
        mesh=mesh,
        out_type=jax.ShapeDtypeStruct((B, D), jnp.float32),
        scratch_types=[
            pltpu.VMEM((CHUNK,), jnp.int32),
            pltpu.VMEM((CHUNK, D), jnp.float32),
            pltpu.SemaphoreType.DMA,
        ],
        compiler_params=pltpu.CompilerParams(use_tc_tiling_on_sc=False),
    )
    def k(lut_hbm, idx_hbm, out_hbm, idx_v, rows_v, sem):
        wid = lax.axis_index("s") * _NC + lax.axis_index("c")
        w_base = wid * b_per_w

        def chunk_body(g, _):
            base = w_base + g * CHUNK
            pltpu.sync_copy(idx_hbm.at[pl.ds(base, CHUNK)], idx_v)
            pltpu.async_copy(lut_hbm.at[idx_v], rows_v, sem).wait()

            def scale_row(r, _):
                for j in range(D // _L):
                    sl = pl.ds(j * _L, _L)
                    rows_v[r, sl] = rows_v[r, sl] * SCALE
                return 0

            lax.fori_loop(0, CHUNK, scale_row, 0)
            pltpu.sync_copy(rows_v, out_hbm.at[pl.ds(base, CHUNK)])
            return 0

        lax.fori_loop(0, n_chunks, chunk_body, 0)

    return k


def kernel(x, lut):
    orig_shape = x.shape
    x_flat = x.reshape(-1).astype(jnp.int32)
    out = _make_gather(x_flat.shape[0], D_MODEL)(lut, x_flat)
    return out.reshape(*orig_shape, D_MODEL)


# speedup vs baseline: 1.1362x; 1.1362x over previous
import functools
import math

import jax
import jax.numpy as jnp
from jax import lax
from jax.experimental import pallas as pl
from jax.experimental.pallas import tpu as pltpu
from jax.experimental.pallas import tpu_sc as plsc

N_TOKEN = 1000000
D_MODEL = 64
SCALE = math.sqrt(D_MODEL)

_info = plsc.get_sparse_core_info()
_NC, _NS, _L = _info.num_cores, _info.num_subcores, _info.num_lanes
_NW = _NC * _NS

CHUNK = 400


def _make_gather(B: int, D: int):
    assert B % (_NW * CHUNK) == 0
    b_per_w = B // _NW
    n_chunks = b_per_w // CHUNK
    assert n_chunks >= 4 and n_chunks % 2 == 0
    mesh = plsc.VectorSubcoreMesh(core_axis_name="c", subcore_axis_name="s")

    @functools.partial(
        pl.kernel,
        mesh=mesh,
        out_type=jax.ShapeDtypeStruct((B, D), jnp.float32),
        scratch_types=[
            pltpu.VMEM((b_per_w,), jnp.int32),
            pltpu.VMEM((2, CHUNK, D), jnp.float32),
            pltpu.VMEM((2, CHUNK, D), jnp.float32),
            pltpu.SemaphoreType.DMA,
            pltpu.SemaphoreType.DMA,
            pltpu.SemaphoreType.DMA,
            pltpu.SemaphoreType.DMA,
        ],
        compiler_params=pltpu.CompilerParams(use_tc_tiling_on_sc=False),
    )
    def k(lut_hbm, idx_hbm, out_hbm, idx_v, rows_in, rows_out,
          sem_g0, sem_g1, sem_o0, sem_o1):
        wid = lax.axis_index("s") * _NC + lax.axis_index("c")
        w_base = wid * b_per_w
        sem_g = (sem_g0, sem_g1)
        sem_o = (sem_o0, sem_o1)

        def gather_pair(g, b):
            return (lut_hbm.at[idx_v.at[pl.ds(g * CHUNK, CHUNK)]],
                    rows_in.at[b], sem_g[b])

        def out_pair(g, b):
            return (rows_out.at[b],
                    out_hbm.at[pl.ds(w_base + g * CHUNK, CHUNK)], sem_o[b])

        def scale_chunk(b):
            rin = rows_in.at[b]
            rout = rows_out.at[b]

            @plsc.parallel_loop(0, CHUNK, unroll=8)
            def _(r):
                for j in range(D // _L):
                    sl = pl.ds(j * _L, _L)
                    rout[r, sl] = rin[r, sl] * SCALE

        def step(g, b, first, last):
            pltpu.make_async_copy(*gather_pair(g, b)).wait()
            if not first:
                pltpu.make_async_copy(*out_pair(g - 2, b)).wait()
            scale_chunk(b)
            pltpu.async_copy(*out_pair(g, b))
            if not last:
                pltpu.async_copy(*gather_pair(g + 2, b))

        pltpu.sync_copy(idx_hbm.at[pl.ds(w_base, b_per_w)], idx_v)

        pltpu.async_copy(*gather_pair(0, 0))
        pltpu.async_copy(*gather_pair(1, 1))
        step(0, 0, True, False)
        step(1, 1, True, False)

        @pl.loop(0, (n_chunks - 4) // 2)
        def _pairs(p):
            g0 = 2 * p + 2
            step(g0, 0, False, False)
            step(g0 + 1, 1, False, False)

        step(n_chunks - 2, 0, False, True)
        step(n_chunks - 1, 1, False, True)
        pltpu.make_async_copy(*out_pair(n_chunks - 2, 0)).wait()
        pltpu.make_async_copy(*out_pair(n_chunks - 1, 1)).wait()

    return k


def kernel(x, lut):
    orig_shape = x.shape
    x_flat = x.reshape(-1).astype(jnp.int32)
    out = _make_gather(x_flat.shape[0], D_MODEL)(lut, x_flat)
    return out.reshape(*orig_shape, D_MODEL)
